# baseline (speedup 1.0000x reference)
import jax
import jax.numpy as jnp
from jax.experimental import pallas as pl
from jax.experimental.pallas import tpu as pltpu

_LRATE = 0.01
_SIZE = 8192
_ROWS = 512

def _body(x_ref, inp_ref, out_ref, y_ref):
    @pl.when(pl.program_id(0) == 0)
    def _():
        _SIGN = jnp.int32(-2147483648)
        _LOW31 = jnp.int32(2147483647)
        x2 = x_ref[...]
        ib = jax.lax.bitcast_convert_type(x2, jnp.int32)
        key = jnp.where(ib >= 0, ib, ib ^ _LOW31)
        rank = jnp.int32((_SIZE - 1) // 2 + 1)

        def step(t, res_b):
            trial_b = res_b | (jnp.int32(1) << (31 - t))
            trial_s = trial_b ^ _SIGN
            cnt = jnp.sum((key < trial_s).astype(jnp.int32))
            return jnp.where(cnt >= rank, res_b, trial_b)

        res_b = jax.lax.fori_loop(0, 32, step, jnp.int32(0))
        med_s = res_b ^ _SIGN
        med_i = jnp.where(med_s >= 0, med_s, med_s ^ _LOW31)
        med_f = jax.lax.bitcast_convert_type(med_i, jnp.float32)
        y_ref[...] = jnp.where(x2 > med_f, jnp.float32(1.0), jnp.float32(0.0))

    a = inp_ref[...] * jnp.float32(-_LRATE)
    out_ref[...] = a * y_ref[...]


def kernel(x, input):
    x2 = x.reshape(1, _SIZE)
    inp2 = input.reshape(_SIZE, 1)
    return pl.pallas_call(
        _body,
        grid=(_SIZE // _ROWS,),
        in_specs=[
            pl.BlockSpec((1, _SIZE), lambda i: (0, 0)),
            pl.BlockSpec((_ROWS, 1), lambda i: (i, 0)),
        ],
        out_specs=pl.BlockSpec((_ROWS, _SIZE), lambda i: (i, 0)),
        out_shape=jax.ShapeDtypeStruct((_SIZE, _SIZE), jnp.float32),
        scratch_shapes=[pltpu.VMEM((1, _SIZE), jnp.float32)],
    )(x2, inp2)


# speedup vs baseline: 1.1002x; 1.1002x over previous
import jax
import jax.numpy as jnp
from jax import lax
from jax.experimental import pallas as pl
from jax.experimental.pallas import tpu as pltpu

_LRATE = 0.01
_SIZE = 8192
_ROWS = 256


def _median_mask(x2):
    _SIGN = jnp.int32(-2147483648)
    _LOW31 = jnp.int32(2147483647)
    ib = lax.bitcast_convert_type(x2, jnp.int32)
    key = jnp.where(ib >= 0, ib, ib ^ _LOW31)
    rank = jnp.int32((_SIZE - 1) // 2 + 1)

    io8 = lax.broadcasted_iota(jnp.int32, (8, 1), 0)
    res_b = jnp.int32(0)
    for r in range(8):
        sh = 28 - 4 * r
        d1 = io8 + 1
        d2 = io8 + 9
        t1 = (res_b + (d1 << sh)) ^ _SIGN
        t2 = (res_b + (d2 << sh)) ^ _SIGN
        c1 = jnp.sum((key < t1).astype(jnp.int32), axis=1, keepdims=True)
        c2 = jnp.sum((key < t2).astype(jnp.int32), axis=1, keepdims=True)
        ind1 = (c1 < rank).astype(jnp.int32)
        ind2 = jnp.where(io8 < 7, (c2 < rank).astype(jnp.int32), 0)
        digit = jnp.sum(ind1) + jnp.sum(ind2)
        res_b = res_b + (digit << sh)
    med_s = res_b ^ _SIGN
    med_i = jnp.where(med_s >= 0, med_s, med_s ^ _LOW31)
    med_f = lax.bitcast_convert_type(med_i, jnp.float32)
    return jnp.where(x2 > med_f, jnp.float32(1.0), jnp.float32(0.0))


def _body(x_ref, inp_ref, out_ref, y_ref):
    @pl.when(pl.program_id(0) == 0)
    def _():
        y_ref[...] = _median_mask(x_ref[...])

    a = inp_ref[...] * jnp.float32(-_LRATE)
    out_ref[...] = lax.dot_general(
        a, y_ref[...], (((0,), (0,)), ((), ())),
        preferred_element_type=jnp.float32,
    )


def kernel(x, input):
    x2 = x.reshape(1, _SIZE)
    inp2 = input.reshape(1, _SIZE)
    return pl.pallas_call(
        _body,
        grid=(_SIZE // _ROWS,),
        in_specs=[
            pl.BlockSpec((1, _SIZE), lambda i: (0, 0)),
            pl.BlockSpec((1, _ROWS), lambda i: (0, i)),
        ],
        out_specs=pl.BlockSpec((_ROWS, _SIZE), lambda i: (i, 0)),
        out_shape=jax.ShapeDtypeStruct((_SIZE, _SIZE), jnp.float32),
        scratch_shapes=[pltpu.VMEM((1, _SIZE), jnp.float32)],
    )(x2, inp2)
